# Initial kernel scaffold; baseline (speedup 1.0000x reference)
import jax
import jax.numpy as jnp
from jax.experimental import pallas as pl


def kernel(in_feat, edge_feat, edge_index, W1, b1, W2, b2):
    raise NotImplementedError("write your pallas kernel here")


# speedup vs baseline: 5.1776x; 5.1776x over previous
import functools

import jax
import jax.numpy as jnp
from jax import lax
from jax.experimental import pallas as pl
from jax.experimental.pallas import tpu as pltpu
from jax.experimental.pallas import tpu_sc as plsc

N = 10000
E = 320000
D = 128
H = 128
C = 40

NC = 2
NS = 16
LANES = 16
NW = NC * NS

N_PAD = 10240
ROWS_T = N_PAD // NS
BB = 80
EPT = E // NW
NBLK = EPT // BB
NCHK = 5
CB = NBLK // NCHK
ECH = CB * BB
EPH = E // NS
C2P = 48

_mesh = plsc.VectorSubcoreMesh(
    core_axis_name="c", subcore_axis_name="s", num_cores=NC, num_subcores=NS)


def _zero_ref(ref, n):
  def body(k, _):
    ref[pl.ds(k * LANES, LANES)] = jnp.zeros((LANES,), ref.dtype)
    return 0
  lax.fori_loop(0, n // LANES, body, 0)


@functools.partial(
    pl.kernel,
    out_type=jax.ShapeDtypeStruct((E,), jnp.float32),
    mesh=_mesh,
    compiler_params=pltpu.CompilerParams(needs_layout_passes=False),
    scratch_types=[
        pltpu.VMEM((EPH,), jnp.int32),
        pltpu.VMEM((N_PAD,), jnp.float32),
        pltpu.VMEM((ROWS_T,), jnp.float32),
        pltpu.VMEM((ROWS_T,), jnp.float32),
        pltpu.VMEM((N_PAD,), jnp.float32),
        pltpu.VMEM((EPT,), jnp.int32),
        pltpu.VMEM((EPT,), jnp.float32),
        pltpu.VMEM((EPT,), jnp.float32),
        pltpu.VMEM_SHARED((NS, N_PAD), jnp.float32),
        pltpu.VMEM_SHARED((N_PAD,), jnp.float32),
    ],
)
def _coeff_kernel(dst_hbm, ef_hbm, c_hbm, dstv, hist, tmp, acc, deg,
                  dst3, efv, cout, hists_sp, deg_sp):
  cid = lax.axis_index("c")
  sid = lax.axis_index("s")

  _zero_ref(hist, N_PAD)
  pltpu.sync_copy(dst_hbm.at[pl.ds(sid * EPH, EPH)], dstv)
  ones = jnp.ones((LANES,), jnp.float32)

  def h_body(k, _):
    dvec = dstv[pl.ds(k * LANES, LANES)]
    plsc.addupdate_scatter(hist, [dvec], ones)
    return 0
  lax.fori_loop(0, EPH // LANES, h_body, 0)
  pltpu.sync_copy(hist, hists_sp.at[sid])
  plsc.subcore_barrier()

  _zero_ref(acc, ROWS_T)
  for j in range(NS):
    pltpu.sync_copy(hists_sp.at[j, pl.ds(sid * ROWS_T, ROWS_T)], tmp)

    def a_body(k, _):
      sl = pl.ds(k * LANES, LANES)
      acc[sl] = acc[sl] + tmp[sl]
      return 0
    lax.fori_loop(0, ROWS_T // LANES, a_body, 0)
  pltpu.sync_copy(acc, deg_sp.at[pl.ds(sid * ROWS_T, ROWS_T)])
  plsc.subcore_barrier()
  pltpu.sync_copy(deg_sp, deg)

  base = (cid * NS + sid) * EPT
  pltpu.sync_copy(dst_hbm.at[pl.ds(base, EPT)], dst3)
  pltpu.sync_copy(ef_hbm.at[pl.ds(base, EPT)], efv)

  def c_body(k, _):
    sl = pl.ds(k * LANES, LANES)
    dvec = dst3[sl]
    g = plsc.load_gather(deg, [dvec])
    cout[sl] = efv[sl] / jnp.maximum(g, 1.0)
    return 0
  lax.fori_loop(0, EPT // LANES, c_body, 0)
  pltpu.sync_copy(cout, c_hbm.at[pl.ds(base, EPT)])


def _make_agg(K):
  @functools.partial(
      pl.kernel,
      out_type=jax.ShapeDtypeStruct((NC, N_PAD, K), jnp.float32),
      mesh=_mesh,
      compiler_params=pltpu.CompilerParams(needs_layout_passes=False),
      scratch_types=[
          pltpu.VMEM((CB, BB), jnp.int32),
          pltpu.VMEM((CB, BB), jnp.int32),
          pltpu.VMEM((ECH,), jnp.float32),
          pltpu.VMEM((BB, K), jnp.float32),
          pltpu.SemaphoreType.DMA,
          pltpu.VMEM_SHARED((N_PAD, K), jnp.float32),
      ],
  )
  def agg(x_hbm, src_hbm, dst_hbm, c_hbm, yp_hbm,
          srcv, dstv, cv, rows, sem, y_sp):
    cid = lax.axis_index("c")
    sid = lax.axis_index("s")
    tid = cid * NS + sid

    def z_body(k, _):
      for j in range(K // LANES):
        rows[k, pl.ds(j * LANES, LANES)] = jnp.zeros((LANES,), jnp.float32)
      return 0
    lax.fori_loop(0, BB, z_body, 0)
    for q in range(ROWS_T // BB):
      pltpu.sync_copy(rows, y_sp.at[pl.ds(sid * ROWS_T + q * BB, BB)])
    plsc.subcore_barrier()

    def chunk_body(ch, _):
      pltpu.sync_copy(src_hbm.at[tid, ch], srcv)
      pltpu.sync_copy(dst_hbm.at[tid, ch], dstv)
      pltpu.sync_copy(c_hbm.at[pl.ds(tid * EPT + ch * ECH, ECH)], cv)

      def blk_body(b, _):
        pltpu.async_copy(x_hbm.at[srcv.at[b]], rows, sem).wait()

        def e_body(i, _):
          ci = plsc.load_gather(cv, [jnp.broadcast_to(b * BB + i, (LANES,))])
          for j in range(K // LANES):
            sl = pl.ds(j * LANES, LANES)
            rows[i, sl] = rows[i, sl] * ci
          return 0
        lax.fori_loop(0, BB, e_body, 0)

        pltpu.sync_copy(rows, y_sp.at[dstv.at[b]], add=True)
        return 0
      lax.fori_loop(0, CB, blk_body, 0)
      return 0
    lax.fori_loop(0, NCHK, chunk_body, 0)
    plsc.subcore_barrier()

    pltpu.sync_copy(y_sp.at[pl.ds(sid * ROWS_T, ROWS_T)],
                    yp_hbm.at[cid, pl.ds(sid * ROWS_T, ROWS_T)])

  return agg


_agg = _make_agg(D)


_BN = 1000


def _tc1_body(x_ref, y_ref, w1_ref, b1_ref, h1_ref):
  x = x_ref[...]
  g = y_ref[0] + y_ref[1]
  w1 = w1_ref[...]
  h = (jnp.dot(x, w1[:D], preferred_element_type=jnp.float32)
       + jnp.dot(g, w1[D:], preferred_element_type=jnp.float32)
       + b1_ref[...])
  h1_ref[...] = jnp.maximum(h, 0.0)


def _tc1(in_feat, y1p, W1, b1):
  return pl.pallas_call(
      _tc1_body,
      grid=(N // _BN,),
      in_specs=[
          pl.BlockSpec((_BN, D), lambda i: (i, 0)),
          pl.BlockSpec((NC, _BN, D), lambda i: (0, i, 0)),
          pl.BlockSpec((2 * D, H), lambda i: (0, 0)),
          pl.BlockSpec((1, H), lambda i: (0, 0)),
      ],
      out_specs=pl.BlockSpec((_BN, H), lambda i: (i, 0)),
      out_shape=jax.ShapeDtypeStruct((N, H), jnp.float32),
  )(in_feat, y1p, W1, b1)


def _tc2_body(h_ref, y_ref, w2t_ref, w2b_ref, b2_ref, out_ref):
  g = y_ref[0] + y_ref[1]
  out_ref[...] = (
      jnp.dot(h_ref[...], w2t_ref[...], preferred_element_type=jnp.float32)
      + jnp.dot(g, w2b_ref[...], preferred_element_type=jnp.float32)
      + b2_ref[...])


def _tc2(h1, y2p, W2top, W2bot, b2):
  return pl.pallas_call(
      _tc2_body,
      grid=(N // _BN,),
      in_specs=[
          pl.BlockSpec((_BN, H), lambda i: (i, 0)),
          pl.BlockSpec((NC, _BN, H), lambda i: (0, i, 0)),
          pl.BlockSpec((H, C2P), lambda i: (0, 0)),
          pl.BlockSpec((H, C2P), lambda i: (0, 0)),
          pl.BlockSpec((1, C2P), lambda i: (0, 0)),
      ],
      out_specs=pl.BlockSpec((_BN, C2P), lambda i: (i, 0)),
      out_shape=jax.ShapeDtypeStruct((N, C2P), jnp.float32),
  )(h1, y2p, W2top, W2bot, b2)


def kernel(in_feat, edge_feat, edge_index, W1, b1, W2, b2):
  ei = edge_index.astype(jnp.int32)
  src = ei[0]
  dst = ei[1]
  ef = edge_feat.reshape(E)
  src_b = src.reshape(NW, NCHK, CB, BB)
  dst_b = dst.reshape(NW, NCHK, CB, BB)

  W2bot = jnp.pad(W2[H:], ((0, 0), (0, C2P - C)))
  W2top = jnp.pad(W2[:H], ((0, 0), (0, C2P - C)))
  b2p = jnp.pad(b2, (0, C2P - C)).reshape(1, C2P)
  b1r = b1.reshape(1, H)

  c = _coeff_kernel(dst, ef)
  y1p = _agg(in_feat, src_b, dst_b, c)
  h1 = _tc1(in_feat, y1p, W1, b1r)
  y2p = _agg(h1, src_b, dst_b, c)
  out = _tc2(h1, y2p, W2top, W2bot, b2p)
  return out[:, :C]


# speedup vs baseline: 8.9218x; 1.7232x over previous
import functools

import jax
import jax.numpy as jnp
from jax import lax
from jax.experimental import pallas as pl
from jax.experimental.pallas import tpu as pltpu
from jax.experimental.pallas import tpu_sc as plsc

N = 10000
E = 320000
D = 128
H = 128
C = 40

NC = 2
NS = 16
LANES = 16
NW = NC * NS

N_PAD = 10240
ROWS_T = N_PAD // NS
BB = 80
EPT = E // NW
NBLK = EPT // BB
NCHK = 5
CB = NBLK // NCHK
ECH = CB * BB
EPH = E // NS
C2P = 48

_mesh = plsc.VectorSubcoreMesh(
    core_axis_name="c", subcore_axis_name="s", num_cores=NC, num_subcores=NS)


def _zero_ref(ref, n):
  def body(k, _):
    ref[pl.ds(k * LANES, LANES)] = jnp.zeros((LANES,), ref.dtype)
    return 0
  lax.fori_loop(0, n // LANES, body, 0)


@functools.partial(
    pl.kernel,
    out_type=jax.ShapeDtypeStruct((E,), jnp.float32),
    mesh=_mesh,
    compiler_params=pltpu.CompilerParams(needs_layout_passes=False),
    scratch_types=[
        pltpu.VMEM((EPH,), jnp.int32),
        pltpu.VMEM((N_PAD,), jnp.float32),
        pltpu.VMEM((ROWS_T,), jnp.float32),
        pltpu.VMEM((ROWS_T,), jnp.float32),
        pltpu.VMEM((N_PAD,), jnp.float32),
        pltpu.VMEM((EPT,), jnp.int32),
        pltpu.VMEM((EPT,), jnp.float32),
        pltpu.VMEM((EPT,), jnp.float32),
        pltpu.VMEM_SHARED((NS, N_PAD), jnp.float32),
        pltpu.VMEM_SHARED((N_PAD,), jnp.float32),
    ],
)
def _coeff_kernel(dst_hbm, ef_hbm, c_hbm, dstv, hist, tmp, acc, deg,
                  dst3, efv, cout, hists_sp, deg_sp):
  cid = lax.axis_index("c")
  sid = lax.axis_index("s")

  _zero_ref(hist, N_PAD)
  pltpu.sync_copy(dst_hbm.at[pl.ds(sid * EPH, EPH)], dstv)
  ones = jnp.ones((LANES,), jnp.float32)

  def h_body(k, _):
    dvec = dstv[pl.ds(k * LANES, LANES)]
    plsc.addupdate_scatter(hist, [dvec], ones)
    return 0
  lax.fori_loop(0, EPH // LANES, h_body, 0)
  pltpu.sync_copy(hist, hists_sp.at[sid])
  plsc.subcore_barrier()

  _zero_ref(acc, ROWS_T)
  for j in range(NS):
    pltpu.sync_copy(hists_sp.at[j, pl.ds(sid * ROWS_T, ROWS_T)], tmp)

    def a_body(k, _):
      sl = pl.ds(k * LANES, LANES)
      acc[sl] = acc[sl] + tmp[sl]
      return 0
    lax.fori_loop(0, ROWS_T // LANES, a_body, 0)
  pltpu.sync_copy(acc, deg_sp.at[pl.ds(sid * ROWS_T, ROWS_T)])
  plsc.subcore_barrier()
  pltpu.sync_copy(deg_sp, deg)

  base = (cid * NS + sid) * EPT
  pltpu.sync_copy(dst_hbm.at[pl.ds(base, EPT)], dst3)
  pltpu.sync_copy(ef_hbm.at[pl.ds(base, EPT)], efv)

  def c_body(k, _):
    sl = pl.ds(k * LANES, LANES)
    dvec = dst3[sl]
    g = plsc.load_gather(deg, [dvec])
    cout[sl] = efv[sl] / jnp.maximum(g, 1.0)
    return 0
  lax.fori_loop(0, EPT // LANES, c_body, 0)
  pltpu.sync_copy(cout, c_hbm.at[pl.ds(base, EPT)])


_NBUF = 3
_ZCH = 1000


def _make_agg(K):
  @functools.partial(
      pl.kernel,
      out_type=jax.ShapeDtypeStruct((NC, N, K), jnp.float32),
      mesh=_mesh,
      compiler_params=pltpu.CompilerParams(needs_layout_passes=False),
      scratch_types=[
          pltpu.VMEM((CB, BB), jnp.int32),
          pltpu.VMEM((CB, BB), jnp.int32),
          pltpu.VMEM((ECH,), jnp.float32),
          [pltpu.VMEM((BB, K), jnp.float32) for _ in range(_NBUF)],
          [pltpu.SemaphoreType.DMA for _ in range(_NBUF)],
          pltpu.VMEM_SHARED((N, K), jnp.float32),
      ],
  )
  def agg(x_hbm, src_hbm, dst_hbm, c_hbm, yp_hbm,
          srcv, dstv, cv, rows, sems, y_sp):
    cid = lax.axis_index("c")
    sid = lax.axis_index("s")
    tid = cid * NS + sid

    def z_body(k, _):
      for j in range(K // LANES):
        rows[0][k, pl.ds(j * LANES, LANES)] = jnp.zeros((LANES,), jnp.float32)
      return 0
    lax.fori_loop(0, BB, z_body, 0)

    @pl.when(sid < N // _ZCH)
    def _():
      for q in range(_ZCH // BB):
        pltpu.sync_copy(rows[0], y_sp.at[pl.ds(sid * _ZCH + q * BB, BB)])
      rem = _ZCH % BB
      if rem:
        pltpu.sync_copy(rows[0].at[pl.ds(0, rem)],
                        y_sp.at[pl.ds(sid * _ZCH + (_ZCH // BB) * BB, rem)])
    plsc.subcore_barrier()

    def chunk_body(ch, _):
      pltpu.sync_copy(src_hbm.at[tid, ch], srcv)
      pltpu.sync_copy(dst_hbm.at[tid, ch], dstv)
      pltpu.sync_copy(c_hbm.at[pl.ds(tid * EPT + ch * ECH, ECH)], cv)

      gat = {}
      scat = {}

      def scale(p):
        buf = p % _NBUF
        gat[p].wait()

        def e_body(i, _):
          ci = plsc.load_gather(cv, [jnp.broadcast_to(p * BB + i, (LANES,))])
          for j in range(K // LANES):
            sl = pl.ds(j * LANES, LANES)
            rows[buf][i, sl] = rows[buf][i, sl] * ci
          return 0
        lax.fori_loop(0, BB, e_body, 0)
        scat[p] = pltpu.async_copy(
            rows[buf], y_sp.at[dstv.at[p]], sems[buf], add=True)

      for b in range(CB):
        buf = b % _NBUF
        if b >= _NBUF:
          scat[b - _NBUF].wait()
        gat[b] = pltpu.async_copy(x_hbm.at[srcv.at[b]], rows[buf], sems[buf])
        if b >= 1:
          scale(b - 1)
      scale(CB - 1)
      for p in range(CB - _NBUF, CB):
        scat[p].wait()
      return 0
    lax.fori_loop(0, NCHK, chunk_body, 0)
    plsc.subcore_barrier()

    @pl.when(sid < N // _ZCH)
    def _():
      pltpu.sync_copy(y_sp.at[pl.ds(sid * _ZCH, _ZCH)],
                      yp_hbm.at[cid, pl.ds(sid * _ZCH, _ZCH)])

  return agg


_agg = _make_agg(D)


_BN = 1000


def _tc1_body(x_ref, y_ref, w1_ref, b1_ref, h1_ref):
  x = x_ref[...]
  g = y_ref[0] + y_ref[1]
  w1 = w1_ref[...]
  h = (jnp.dot(x, w1[:D], preferred_element_type=jnp.float32)
       + jnp.dot(g, w1[D:], preferred_element_type=jnp.float32)
       + b1_ref[...])
  h1_ref[...] = jnp.maximum(h, 0.0)


def _tc1(in_feat, y1p, W1, b1):
  return pl.pallas_call(
      _tc1_body,
      grid=(N // _BN,),
      in_specs=[
          pl.BlockSpec((_BN, D), lambda i: (i, 0)),
          pl.BlockSpec((NC, _BN, D), lambda i: (0, i, 0)),
          pl.BlockSpec((2 * D, H), lambda i: (0, 0)),
          pl.BlockSpec((1, H), lambda i: (0, 0)),
      ],
      out_specs=pl.BlockSpec((_BN, H), lambda i: (i, 0)),
      out_shape=jax.ShapeDtypeStruct((N, H), jnp.float32),
  )(in_feat, y1p, W1, b1)


def _tc2_body(h_ref, y_ref, w2t_ref, w2b_ref, b2_ref, out_ref):
  g = y_ref[0] + y_ref[1]
  out_ref[...] = (
      jnp.dot(h_ref[...], w2t_ref[...], preferred_element_type=jnp.float32)
      + jnp.dot(g, w2b_ref[...], preferred_element_type=jnp.float32)
      + b2_ref[...])


def _tc2(h1, y2p, W2top, W2bot, b2):
  return pl.pallas_call(
      _tc2_body,
      grid=(N // _BN,),
      in_specs=[
          pl.BlockSpec((_BN, H), lambda i: (i, 0)),
          pl.BlockSpec((NC, _BN, H), lambda i: (0, i, 0)),
          pl.BlockSpec((H, C2P), lambda i: (0, 0)),
          pl.BlockSpec((H, C2P), lambda i: (0, 0)),
          pl.BlockSpec((1, C2P), lambda i: (0, 0)),
      ],
      out_specs=pl.BlockSpec((_BN, C2P), lambda i: (i, 0)),
      out_shape=jax.ShapeDtypeStruct((N, C2P), jnp.float32),
  )(h1, y2p, W2top, W2bot, b2)


def kernel(in_feat, edge_feat, edge_index, W1, b1, W2, b2):
  ei = edge_index.astype(jnp.int32)
  src = ei[0]
  dst = ei[1]
  ef = edge_feat.reshape(E)
  src_b = src.reshape(NW, NCHK, CB, BB)
  dst_b = dst.reshape(NW, NCHK, CB, BB)

  W2bot = jnp.pad(W2[H:], ((0, 0), (0, C2P - C)))
  W2top = jnp.pad(W2[:H], ((0, 0), (0, C2P - C)))
  b2p = jnp.pad(b2, (0, C2P - C)).reshape(1, C2P)
  b1r = b1.reshape(1, H)

  c = _coeff_kernel(dst, ef)
  y1p = _agg(in_feat, src_b, dst_b, c)
  h1 = _tc1(in_feat, y1p, W1, b1r)
  y2p = _agg(h1, src_b, dst_b, c)
  out = _tc2(h1, y2p, W2top, W2bot, b2p)
  return out[:, :C]


# speedup vs baseline: 9.7470x; 1.0925x over previous
import functools

import jax
import jax.numpy as jnp
from jax import lax
from jax.experimental import pallas as pl
from jax.experimental.pallas import tpu as pltpu
from jax.experimental.pallas import tpu_sc as plsc

N = 10000
E = 320000
D = 128
H = 128
C = 40

NC = 2
NS = 16
LANES = 16
NW = NC * NS

N_PAD = 10240
ROWS_T = N_PAD // NS
BB = 80
EPT = E // NW
NBLK = EPT // BB
NCHK = 5
CB = NBLK // NCHK
ECH = CB * BB
EPH = E // NS
C2P = 48

_mesh = plsc.VectorSubcoreMesh(
    core_axis_name="c", subcore_axis_name="s", num_cores=NC, num_subcores=NS)


def _zero_ref(ref, n):
  def body(k, _):
    ref[pl.ds(k * LANES, LANES)] = jnp.zeros((LANES,), ref.dtype)
    return 0
  lax.fori_loop(0, n // LANES, body, 0)


@functools.partial(
    pl.kernel,
    out_type=jax.ShapeDtypeStruct((E,), jnp.float32),
    mesh=_mesh,
    compiler_params=pltpu.CompilerParams(needs_layout_passes=False),
    scratch_types=[
        pltpu.VMEM((EPH,), jnp.int32),
        pltpu.VMEM((N_PAD,), jnp.float32),
        pltpu.VMEM((ROWS_T,), jnp.float32),
        pltpu.VMEM((ROWS_T,), jnp.float32),
        pltpu.VMEM((N_PAD,), jnp.float32),
        pltpu.VMEM((EPT,), jnp.int32),
        pltpu.VMEM((EPT,), jnp.float32),
        pltpu.VMEM((EPT,), jnp.float32),
        pltpu.VMEM_SHARED((NS, N_PAD), jnp.float32),
        pltpu.VMEM_SHARED((N_PAD,), jnp.float32),
    ],
)
def _coeff_kernel(dst_hbm, ef_hbm, c_hbm, dstv, hist, tmp, acc, deg,
                  dst3, efv, cout, hists_sp, deg_sp):
  cid = lax.axis_index("c")
  sid = lax.axis_index("s")

  _zero_ref(hist, N_PAD)
  pltpu.sync_copy(dst_hbm.at[pl.ds(sid * EPH, EPH)], dstv)
  ones = jnp.ones((LANES,), jnp.float32)

  def h_body(k, _):
    dvec = dstv[pl.ds(k * LANES, LANES)]
    plsc.addupdate_scatter(hist, [dvec], ones)
    return 0
  lax.fori_loop(0, EPH // LANES, h_body, 0)
  pltpu.sync_copy(hist, hists_sp.at[sid])
  plsc.subcore_barrier()

  _zero_ref(acc, ROWS_T)
  for j in range(NS):
    pltpu.sync_copy(hists_sp.at[j, pl.ds(sid * ROWS_T, ROWS_T)], tmp)

    def a_body(k, _):
      sl = pl.ds(k * LANES, LANES)
      acc[sl] = acc[sl] + tmp[sl]
      return 0
    lax.fori_loop(0, ROWS_T // LANES, a_body, 0)
  pltpu.sync_copy(acc, deg_sp.at[pl.ds(sid * ROWS_T, ROWS_T)])
  plsc.subcore_barrier()
  pltpu.sync_copy(deg_sp, deg)

  base = (cid * NS + sid) * EPT
  pltpu.sync_copy(dst_hbm.at[pl.ds(base, EPT)], dst3)
  pltpu.sync_copy(ef_hbm.at[pl.ds(base, EPT)], efv)

  def c_body(k, _):
    sl = pl.ds(k * LANES, LANES)
    dvec = dst3[sl]
    g = plsc.load_gather(deg, [dvec])
    cout[sl] = efv[sl] / jnp.maximum(g, 1.0)
    return 0
  lax.fori_loop(0, EPT // LANES, c_body, 0)
  pltpu.sync_copy(cout, c_hbm.at[pl.ds(base, EPT)])


_NBUF = 3
_ZCH = 1000


def _make_agg(K):
  @functools.partial(
      pl.kernel,
      out_type=jax.ShapeDtypeStruct((NC, N, K), jnp.float32),
      mesh=_mesh,
      compiler_params=pltpu.CompilerParams(needs_layout_passes=False),
      scratch_types=[
          pltpu.VMEM((CB, BB), jnp.int32),
          pltpu.VMEM((CB, BB), jnp.int32),
          pltpu.VMEM((ECH,), jnp.float32),
          [pltpu.VMEM((BB, K), jnp.float32) for _ in range(_NBUF)],
          [pltpu.SemaphoreType.DMA for _ in range(_NBUF)],
          pltpu.VMEM_SHARED((N, K), jnp.float32),
      ],
  )
  def agg(x_hbm, src_hbm, dst_hbm, c_hbm, yp_hbm,
          srcv, dstv, cv, rows, sems, y_sp):
    cid = lax.axis_index("c")
    sid = lax.axis_index("s")
    tid = cid * NS + sid

    def z_body(k, _):
      for j in range(K // LANES):
        rows[0][k, pl.ds(j * LANES, LANES)] = jnp.zeros((LANES,), jnp.float32)
      return 0
    lax.fori_loop(0, BB, z_body, 0)

    @pl.when(sid < N // _ZCH)
    def _():
      for q in range(_ZCH // BB):
        pltpu.sync_copy(rows[0], y_sp.at[pl.ds(sid * _ZCH + q * BB, BB)])
      rem = _ZCH % BB
      if rem:
        pltpu.sync_copy(rows[0].at[pl.ds(0, rem)],
                        y_sp.at[pl.ds(sid * _ZCH + (_ZCH // BB) * BB, rem)])
    plsc.subcore_barrier()

    def chunk_body(ch, _):
      pltpu.sync_copy(src_hbm.at[tid, ch], srcv)
      pltpu.sync_copy(dst_hbm.at[tid, ch], dstv)
      pltpu.sync_copy(c_hbm.at[pl.ds(tid * EPT + ch * ECH, ECH)], cv)

      gat = {}
      scat = {}

      def scale(p):
        buf = p % _NBUF
        gat[p].wait()

        dnums = lax.GatherDimensionNumbers(
            offset_dims=(), collapsed_slice_dims=(0,), start_index_map=(0,))

        def g_body(g, _):
          cs = cv[pl.ds(p * BB + g * LANES, LANES)]
          for e in range(LANES):
            ci = lax.gather(
                cs, jnp.full((LANES, 1), e, jnp.int32), dnums, (1,),
                mode=lax.GatherScatterMode.PROMISE_IN_BOUNDS)
            i = g * LANES + e
            for j in range(K // LANES):
              sl = pl.ds(j * LANES, LANES)
              rows[buf][i, sl] = rows[buf][i, sl] * ci
          return 0
        lax.fori_loop(0, BB // LANES, g_body, 0)
        scat[p] = pltpu.async_copy(
            rows[buf], y_sp.at[dstv.at[p]], sems[buf], add=True)

      for b in range(CB):
        buf = b % _NBUF
        if b >= _NBUF:
          scat[b - _NBUF].wait()
        gat[b] = pltpu.async_copy(x_hbm.at[srcv.at[b]], rows[buf], sems[buf])
        if b >= 1:
          scale(b - 1)
      scale(CB - 1)
      for p in range(CB - _NBUF, CB):
        scat[p].wait()
      return 0
    lax.fori_loop(0, NCHK, chunk_body, 0)
    plsc.subcore_barrier()

    @pl.when(sid < N // _ZCH)
    def _():
      pltpu.sync_copy(y_sp.at[pl.ds(sid * _ZCH, _ZCH)],
                      yp_hbm.at[cid, pl.ds(sid * _ZCH, _ZCH)])

  return agg


_agg = _make_agg(D)


_BN = 1000


def _mmb_body(x_ref, w_ref, b_ref, o_ref):
  o_ref[...] = (jnp.dot(x_ref[...], w_ref[...],
                        preferred_element_type=jnp.float32) + b_ref[...])


def _mmb(x, w, b):
  kw = w.shape[1]
  return pl.pallas_call(
      _mmb_body,
      grid=(N // _BN,),
      in_specs=[
          pl.BlockSpec((_BN, x.shape[1]), lambda i: (i, 0)),
          pl.BlockSpec(w.shape, lambda i: (0, 0)),
          pl.BlockSpec((1, kw), lambda i: (0, 0)),
      ],
      out_specs=pl.BlockSpec((_BN, kw), lambda i: (i, 0)),
      out_shape=jax.ShapeDtypeStruct((N, kw), jnp.float32),
  )(x, w, b)


def _make_comb(relu):
  def body(p_ref, y_ref, w_ref, o_ref):
    g = y_ref[0] + y_ref[1]
    o = p_ref[...] + jnp.dot(g, w_ref[...],
                             preferred_element_type=jnp.float32)
    o_ref[...] = jnp.maximum(o, 0.0) if relu else o
  return body


def _comb(p, yp, w, relu):
  kw = w.shape[1]
  return pl.pallas_call(
      _make_comb(relu),
      grid=(N // _BN,),
      in_specs=[
          pl.BlockSpec((_BN, kw), lambda i: (i, 0)),
          pl.BlockSpec((NC, _BN, D), lambda i: (0, i, 0)),
          pl.BlockSpec(w.shape, lambda i: (0, 0)),
      ],
      out_specs=pl.BlockSpec((_BN, kw), lambda i: (i, 0)),
      out_shape=jax.ShapeDtypeStruct((N, kw), jnp.float32),
  )(p, yp, w)


def kernel(in_feat, edge_feat, edge_index, W1, b1, W2, b2):
  ei = edge_index.astype(jnp.int32)
  src = ei[0]
  dst = ei[1]
  ef = edge_feat.reshape(E)
  src_b = src.reshape(NW, NCHK, CB, BB)
  dst_b = dst.reshape(NW, NCHK, CB, BB)

  W2bot = jnp.pad(W2[H:], ((0, 0), (0, C2P - C)))
  W2top = jnp.pad(W2[:H], ((0, 0), (0, C2P - C)))
  b2p = jnp.pad(b2, (0, C2P - C)).reshape(1, C2P)
  b1r = b1.reshape(1, H)

  c = _coeff_kernel(dst, ef)
  p1 = _mmb(in_feat, W1[:D], b1r)
  y1p = _agg(in_feat, src_b, dst_b, c)
  h1 = _comb(p1, y1p, W1[D:], relu=True)
  p2 = _mmb(h1, W2top, b2p)
  y2p = _agg(h1, src_b, dst_b, c)
  out = _comb(p2, y2p, W2bot, relu=False)
  return out[:, :C]
